# Initial kernel scaffold; baseline (speedup 1.0000x reference)
import functools
import math

import jax
import jax.numpy as jnp
from jax import lax
from jax.experimental import pallas as pl
from jax.experimental.pallas import tpu as pltpu
import jax.experimental.pallas.tpu_sc as plsc

N = 10000
E = 320000
D = 128
NB = 128
L = 6
NG = 64
NATOM = 64
MAXR = 5.0
AVG_DEG = 32.0

NC = 2
NS = 16
NW = NC * NS
EPT = E // NW
C = 80
K = EPT // C
RPT = N // NS

_mesh = plsc.VectorSubcoreMesh(core_axis_name="c", subcore_axis_name="s")


@functools.partial(
    pl.kernel,
    out_type=jax.ShapeDtypeStruct((E,), jnp.float32),
    mesh=_mesh,
    scratch_types=[
        pltpu.VMEM((4 * N,), jnp.float32),
        pltpu.VMEM((EPT,), jnp.int32),
        pltpu.VMEM((EPT,), jnp.int32),
        pltpu.VMEM((EPT,), jnp.float32),
    ],
)
def _sc_d2(posf_hbm, src_hbm, dst_hbm, out_hbm, pos_v, src_v, dst_v, d2_v):
    c = lax.axis_index("c")
    s = lax.axis_index("s")
    w = c * NS + s
    base = w * EPT
    pltpu.sync_copy(posf_hbm, pos_v)
    pltpu.sync_copy(src_hbm.at[pl.ds(base, EPT)], src_v)
    pltpu.sync_copy(dst_hbm.at[pl.ds(base, EPT)], dst_v)

    def body(t, _):
        sl = pl.ds(t * 16, 16)
        si = src_v[sl] * 4
        di = dst_v[sl] * 4
        dx = plsc.load_gather(pos_v, [si]) - plsc.load_gather(pos_v, [di])
        dy = plsc.load_gather(pos_v, [si + 1]) - plsc.load_gather(pos_v, [di + 1])
        dz = plsc.load_gather(pos_v, [si + 2]) - plsc.load_gather(pos_v, [di + 2])
        d2_v[sl] = dx * dx + dy * dy + dz * dz
        return 0

    lax.fori_loop(0, EPT // 16, body, 0)
    pltpu.sync_copy(d2_v, out_hbm.at[pl.ds(base, EPT)])


@functools.partial(
    pl.kernel,
    out_type=jax.ShapeDtypeStruct((NC, N, D), jnp.float32),
    mesh=_mesh,
    scratch_types=[
        pltpu.VMEM_SHARED((N, D), jnp.float32),
        pltpu.VMEM((K, C), jnp.int32),
        pltpu.VMEM((K, C), jnp.int32),
        pltpu.VMEM((C, D), jnp.float32),
        pltpu.VMEM((C, D), jnp.float32),
        pltpu.SemaphoreType.DMA,
    ],
)
def _sc_agg(h_hbm, ew_hbm, src_hbm, dst_hbm, zeros_hbm, out_hbm,
            agg_sh, src_v, dst_v, msg_v, ew_v, sem):
    c = lax.axis_index("c")
    s = lax.axis_index("s")
    w = c * NS + s
    row0 = s * RPT
    pltpu.sync_copy(zeros_hbm, agg_sh.at[pl.ds(row0, RPT)])
    pltpu.sync_copy(src_hbm.at[pl.ds(w * K, K)], src_v)
    pltpu.sync_copy(dst_hbm.at[pl.ds(w * K, K)], dst_v)
    plsc.subcore_barrier()

    def chunk(k, _):
        pltpu.async_copy(h_hbm.at[src_v.at[k]], msg_v, sem).wait()
        pltpu.sync_copy(ew_hbm.at[pl.ds((w * K + k) * C, C)], ew_v)

        def rowbody(r, _):
            for j in range(D // 16):
                sl = pl.ds(j * 16, 16)
                msg_v[r, sl] = msg_v[r, sl] * ew_v[r, sl]
            return 0

        lax.fori_loop(0, C, rowbody, 0)
        pltpu.sync_copy(msg_v, agg_sh.at[dst_v.at[k]], add=True)
        return 0

    lax.fori_loop(0, K, chunk, 0)
    plsc.subcore_barrier()
    pltpu.sync_copy(agg_sh.at[pl.ds(row0, RPT)],
                    out_hbm.at[c, pl.ds(row0, RPT)])


_RB = 2000
_GN = N // _RB
_EB = 2000
_GE = E // _EB


def _tc_ew_body(d2_ref, cen_ref, w1_ref, w2_ref, out_ref):
    d2 = d2_ref[...]
    dist = jnp.sqrt(d2 + 1e-8)
    cen = cen_ref[...]
    width = MAXR / NB
    rbf = jnp.exp(-((dist - cen) ** 2) * (1.0 / (2.0 * width * width)))
    h1 = jax.nn.silu(jnp.dot(rbf, w1_ref[...], preferred_element_type=jnp.float32))
    out_ref[...] = jnp.dot(h1, w2_ref[...], preferred_element_type=jnp.float32)


def _tc_ew(d2, centers, w1, w2):
    return pl.pallas_call(
        _tc_ew_body,
        grid=(_GE,),
        in_specs=[
            pl.BlockSpec((_EB, 1), lambda i: (i, 0)),
            pl.BlockSpec((1, NB), lambda i: (0, 0)),
            pl.BlockSpec((NB, 64), lambda i: (0, 0)),
            pl.BlockSpec((64, D), lambda i: (0, 0)),
        ],
        out_specs=pl.BlockSpec((_EB, D), lambda i: (i, 0)),
        out_shape=jax.ShapeDtypeStruct((E, D), jnp.float32),
    )(d2, centers, w1, w2)


def _layernorm(x, w, b):
    mu = jnp.mean(x, axis=-1, keepdims=True)
    var = jnp.var(x, axis=-1, keepdims=True)
    return (x - mu) / jnp.sqrt(var + 1e-5) * w + b


def _tc_embed_body(na_ref, tab_ref, lnw_ref, lnb_ref, x_ref, h_ref):
    na = na_ref[...]
    ids = lax.broadcasted_iota(jnp.int32, (_RB, NATOM), 1)
    onehot = (na == ids).astype(jnp.float32)
    x = jnp.dot(onehot, tab_ref[...], preferred_element_type=jnp.float32)
    x_ref[...] = x
    h_ref[...] = _layernorm(x, lnw_ref[...], lnb_ref[...])


def _tc_embed(na, tab, lnw0, lnb0):
    return pl.pallas_call(
        _tc_embed_body,
        grid=(_GN,),
        in_specs=[
            pl.BlockSpec((_RB, 1), lambda i: (i, 0)),
            pl.BlockSpec((NATOM, D), lambda i: (0, 0)),
            pl.BlockSpec((1, D), lambda i: (0, 0)),
            pl.BlockSpec((1, D), lambda i: (0, 0)),
        ],
        out_specs=[
            pl.BlockSpec((_RB, D), lambda i: (i, 0)),
            pl.BlockSpec((_RB, D), lambda i: (i, 0)),
        ],
        out_shape=[
            jax.ShapeDtypeStruct((N, D), jnp.float32),
            jax.ShapeDtypeStruct((N, D), jnp.float32),
        ],
    )(na, tab, lnw0, lnb0)


def _tc_update_body(x_ref, agg_ref, w_ref, lnw_ref, lnb_ref, xn_ref, hn_ref):
    agg = (agg_ref[0] + agg_ref[1]) * (1.0 / math.sqrt(AVG_DEG))
    up = jnp.dot(agg, w_ref[...], preferred_element_type=jnp.float32)
    xn = x_ref[...] + jax.nn.silu(up)
    xn_ref[...] = xn
    hn_ref[...] = _layernorm(xn, lnw_ref[...], lnb_ref[...])


def _tc_update(x, aggp, w, lnw, lnb):
    return pl.pallas_call(
        _tc_update_body,
        grid=(_GN,),
        in_specs=[
            pl.BlockSpec((_RB, D), lambda i: (i, 0)),
            pl.BlockSpec((NC, _RB, D), lambda i: (0, i, 0)),
            pl.BlockSpec((D, D), lambda i: (0, 0)),
            pl.BlockSpec((1, D), lambda i: (0, 0)),
            pl.BlockSpec((1, D), lambda i: (0, 0)),
        ],
        out_specs=[
            pl.BlockSpec((_RB, D), lambda i: (i, 0)),
            pl.BlockSpec((_RB, D), lambda i: (i, 0)),
        ],
        out_shape=[
            jax.ShapeDtypeStruct((N, D), jnp.float32),
            jax.ShapeDtypeStruct((N, D), jnp.float32),
        ],
    )(x, aggp, w, lnw, lnb)


def _tc_head_body(x_ref, wh_ref, b_ref, out_ref):
    i = pl.program_id(0)

    @pl.when(i == 0)
    def _():
        out_ref[...] = jnp.zeros_like(out_ref)

    e_node = jnp.dot(x_ref[...], wh_ref[...], preferred_element_type=jnp.float32)
    ids = lax.broadcasted_iota(jnp.int32, (_RB, NG), 1)
    onehot = (b_ref[...] == ids).astype(jnp.float32)
    out_ref[...] += lax.dot_general(
        onehot, e_node, (((0,), (0,)), ((), ())),
        preferred_element_type=jnp.float32)


def _tc_head(x, wh, batch2):
    return pl.pallas_call(
        _tc_head_body,
        grid=(_GN,),
        in_specs=[
            pl.BlockSpec((_RB, D), lambda i: (i, 0)),
            pl.BlockSpec((D, 1), lambda i: (0, 0)),
            pl.BlockSpec((_RB, 1), lambda i: (i, 0)),
        ],
        out_specs=pl.BlockSpec((NG, 1), lambda i: (0, 0)),
        out_shape=jax.ShapeDtypeStruct((NG, 1), jnp.float32),
    )(x, wh, batch2)


def kernel(pos, node_atom, batch, edge_index, atom_table, rbf_centers,
           W_rbf1, W_rbf2, W_layers, ln_w, ln_b, W_head):
    src = edge_index[0].astype(jnp.int32)
    dst = edge_index[1].astype(jnp.int32)
    src2 = src.reshape(NW * K, C)
    dst2 = dst.reshape(NW * K, C)
    posf = jnp.pad(pos, ((0, 0), (0, 1))).reshape(-1)
    zeros = jnp.zeros((RPT, D), jnp.float32)
    na2 = node_atom.astype(jnp.int32).reshape(N, 1)
    b2 = batch.astype(jnp.int32).reshape(N, 1)

    d2 = _sc_d2(posf, src, dst)
    ew = _tc_ew(d2.reshape(E, 1), rbf_centers.reshape(1, NB), W_rbf1, W_rbf2)

    x, h = _tc_embed(na2, atom_table, ln_w[0].reshape(1, D), ln_b[0].reshape(1, D))
    for l in range(L):
        aggp = _sc_agg(h, ew, src2, dst2, zeros)
        nl = (l + 1) % L
        x, h = _tc_update(x, aggp, W_layers[l],
                          ln_w[nl].reshape(1, D), ln_b[nl].reshape(1, D))
    return _tc_head(x, W_head, b2)


# speedup vs baseline: 2.9421x; 2.9421x over previous
import functools
import math

import jax
import jax.numpy as jnp
from jax import lax
from jax.experimental import pallas as pl
from jax.experimental.pallas import tpu as pltpu
import jax.experimental.pallas.tpu_sc as plsc

N = 10000
E = 320000
D = 128
NB = 128
L = 6
NG = 64
NATOM = 64
MAXR = 5.0
AVG_DEG = 32.0

NC = 2
NS = 16
NW = NC * NS
EPT = E // NW
C = 80
K = EPT // C
NP_ = 10240
RPT = NP_ // NS

_mesh = plsc.VectorSubcoreMesh(core_axis_name="c", subcore_axis_name="s")


@functools.partial(
    pl.kernel,
    out_type=jax.ShapeDtypeStruct((E,), jnp.float32),
    mesh=_mesh,
    scratch_types=[
        pltpu.VMEM((C,), jnp.int32),
        pltpu.VMEM((C,), jnp.int32),
        pltpu.VMEM((6, C), jnp.float32),
        pltpu.VMEM((EPT,), jnp.float32),
        pltpu.SemaphoreType.DMA,
    ],
)
def _sc_d2(px_hbm, py_hbm, pz_hbm, src_hbm, dst_hbm, out_hbm,
           src_v, dst_v, g_v, d2_v, sem):
    c = lax.axis_index("c")
    s = lax.axis_index("s")
    w = c * NS + s
    base = w * EPT

    def chunk(k, _):
        pltpu.sync_copy(src_hbm.at[pl.ds(base + k * C, C)], src_v)
        pltpu.sync_copy(dst_hbm.at[pl.ds(base + k * C, C)], dst_v)
        cps = [
            pltpu.async_copy(px_hbm.at[src_v], g_v.at[0], sem),
            pltpu.async_copy(py_hbm.at[src_v], g_v.at[1], sem),
            pltpu.async_copy(pz_hbm.at[src_v], g_v.at[2], sem),
            pltpu.async_copy(px_hbm.at[dst_v], g_v.at[3], sem),
            pltpu.async_copy(py_hbm.at[dst_v], g_v.at[4], sem),
            pltpu.async_copy(pz_hbm.at[dst_v], g_v.at[5], sem),
        ]
        for cp in cps:
            cp.wait()
        for t in range(C // 16):
            sl = pl.ds(t * 16, 16)
            dx = g_v[0, sl] - g_v[3, sl]
            dy = g_v[1, sl] - g_v[4, sl]
            dz = g_v[2, sl] - g_v[5, sl]
            d2_v[pl.ds(k * C + t * 16, 16)] = dx * dx + dy * dy + dz * dz
        return 0

    lax.fori_loop(0, K, chunk, 0)
    pltpu.sync_copy(d2_v, out_hbm.at[pl.ds(base, EPT)])


@functools.partial(
    pl.kernel,
    out_type=jax.ShapeDtypeStruct((NC, NP_, D), jnp.float32),
    mesh=_mesh,
    scratch_types=[
        pltpu.VMEM_SHARED((NP_, D), jnp.float32),
        pltpu.VMEM((C,), jnp.int32),
        pltpu.VMEM((C,), jnp.int32),
        pltpu.VMEM((C, D), jnp.float32),
        pltpu.VMEM((C, D), jnp.float32),
        pltpu.SemaphoreType.DMA,
    ],
)
def _sc_agg(h_hbm, ew_hbm, src_hbm, dst_hbm, zeros_hbm, out_hbm,
            agg_sh, src_v, dst_v, msg_v, ew_v, sem):
    c = lax.axis_index("c")
    s = lax.axis_index("s")
    w = c * NS + s
    base = w * EPT
    row0 = s * RPT
    pltpu.sync_copy(zeros_hbm, agg_sh.at[pl.ds(row0, RPT)])
    plsc.subcore_barrier()

    def chunk(k, _):
        pltpu.sync_copy(src_hbm.at[pl.ds(base + k * C, C)], src_v)
        pltpu.sync_copy(dst_hbm.at[pl.ds(base + k * C, C)], dst_v)
        pltpu.async_copy(h_hbm.at[src_v], msg_v, sem).wait()
        pltpu.sync_copy(ew_hbm.at[pl.ds(base + k * C, C)], ew_v)

        def rowbody(r, _):
            for j in range(D // 16):
                sl = pl.ds(j * 16, 16)
                msg_v[r, sl] = msg_v[r, sl] * ew_v[r, sl]
            return 0

        lax.fori_loop(0, C, rowbody, 0)
        pltpu.sync_copy(msg_v, agg_sh.at[dst_v], add=True)
        return 0

    lax.fori_loop(0, K, chunk, 0)
    plsc.subcore_barrier()
    pltpu.sync_copy(agg_sh.at[pl.ds(row0, RPT)],
                    out_hbm.at[c, pl.ds(row0, RPT)])


_RB = 2000
_GN = N // _RB
_EB = 2000
_GE = E // _EB


def _tc_ew_body(d2_ref, cen_ref, w1_ref, w2_ref, out_ref):
    d2 = d2_ref[...]
    dist = jnp.sqrt(d2 + 1e-8)
    cen = cen_ref[...]
    width = MAXR / NB
    rbf = jnp.exp(-((dist - cen) ** 2) * (1.0 / (2.0 * width * width)))
    h1 = jax.nn.silu(jnp.dot(rbf, w1_ref[...], preferred_element_type=jnp.float32))
    out_ref[...] = jnp.dot(h1, w2_ref[...], preferred_element_type=jnp.float32)


def _tc_ew(d2, centers, w1, w2):
    return pl.pallas_call(
        _tc_ew_body,
        grid=(_GE,),
        in_specs=[
            pl.BlockSpec((_EB, 1), lambda i: (i, 0)),
            pl.BlockSpec((1, NB), lambda i: (0, 0)),
            pl.BlockSpec((NB, 64), lambda i: (0, 0)),
            pl.BlockSpec((64, D), lambda i: (0, 0)),
        ],
        out_specs=pl.BlockSpec((_EB, D), lambda i: (i, 0)),
        out_shape=jax.ShapeDtypeStruct((E, D), jnp.float32),
    )(d2, centers, w1, w2)


def _layernorm(x, w, b):
    mu = jnp.mean(x, axis=-1, keepdims=True)
    var = jnp.var(x, axis=-1, keepdims=True)
    return (x - mu) / jnp.sqrt(var + 1e-5) * w + b


def _tc_embed_body(na_ref, tab_ref, lnw_ref, lnb_ref, x_ref, h_ref):
    na = na_ref[...]
    ids = lax.broadcasted_iota(jnp.int32, (_RB, NATOM), 1)
    onehot = (na == ids).astype(jnp.float32)
    x = jnp.dot(onehot, tab_ref[...], preferred_element_type=jnp.float32)
    x_ref[...] = x
    h_ref[...] = _layernorm(x, lnw_ref[...], lnb_ref[...])


def _tc_embed(na, tab, lnw0, lnb0):
    return pl.pallas_call(
        _tc_embed_body,
        grid=(_GN,),
        in_specs=[
            pl.BlockSpec((_RB, 1), lambda i: (i, 0)),
            pl.BlockSpec((NATOM, D), lambda i: (0, 0)),
            pl.BlockSpec((1, D), lambda i: (0, 0)),
            pl.BlockSpec((1, D), lambda i: (0, 0)),
        ],
        out_specs=[
            pl.BlockSpec((_RB, D), lambda i: (i, 0)),
            pl.BlockSpec((_RB, D), lambda i: (i, 0)),
        ],
        out_shape=[
            jax.ShapeDtypeStruct((N, D), jnp.float32),
            jax.ShapeDtypeStruct((N, D), jnp.float32),
        ],
    )(na, tab, lnw0, lnb0)


def _tc_update_body(x_ref, agg_ref, w_ref, lnw_ref, lnb_ref, xn_ref, hn_ref):
    agg = (agg_ref[0] + agg_ref[1]) * (1.0 / math.sqrt(AVG_DEG))
    up = jnp.dot(agg, w_ref[...], preferred_element_type=jnp.float32)
    xn = x_ref[...] + jax.nn.silu(up)
    xn_ref[...] = xn
    hn_ref[...] = _layernorm(xn, lnw_ref[...], lnb_ref[...])


def _tc_update(x, aggp, w, lnw, lnb):
    return pl.pallas_call(
        _tc_update_body,
        grid=(_GN,),
        in_specs=[
            pl.BlockSpec((_RB, D), lambda i: (i, 0)),
            pl.BlockSpec((NC, _RB, D), lambda i: (0, i, 0)),
            pl.BlockSpec((D, D), lambda i: (0, 0)),
            pl.BlockSpec((1, D), lambda i: (0, 0)),
            pl.BlockSpec((1, D), lambda i: (0, 0)),
        ],
        out_specs=[
            pl.BlockSpec((_RB, D), lambda i: (i, 0)),
            pl.BlockSpec((_RB, D), lambda i: (i, 0)),
        ],
        out_shape=[
            jax.ShapeDtypeStruct((N, D), jnp.float32),
            jax.ShapeDtypeStruct((N, D), jnp.float32),
        ],
    )(x, aggp, w, lnw, lnb)


def _tc_head_body(x_ref, wh_ref, b_ref, out_ref):
    i = pl.program_id(0)

    @pl.when(i == 0)
    def _():
        out_ref[...] = jnp.zeros_like(out_ref)

    e_node = jnp.dot(x_ref[...], wh_ref[...], preferred_element_type=jnp.float32)
    ids = lax.broadcasted_iota(jnp.int32, (_RB, NG), 1)
    onehot = (b_ref[...] == ids).astype(jnp.float32)
    out_ref[...] += lax.dot_general(
        onehot, e_node, (((0,), (0,)), ((), ())),
        preferred_element_type=jnp.float32)


def _tc_head(x, wh, batch2):
    return pl.pallas_call(
        _tc_head_body,
        grid=(_GN,),
        in_specs=[
            pl.BlockSpec((_RB, D), lambda i: (i, 0)),
            pl.BlockSpec((D, 1), lambda i: (0, 0)),
            pl.BlockSpec((_RB, 1), lambda i: (i, 0)),
        ],
        out_specs=pl.BlockSpec((NG, 1), lambda i: (0, 0)),
        out_shape=jax.ShapeDtypeStruct((NG, 1), jnp.float32),
    )(x, wh, batch2)


def kernel(pos, node_atom, batch, edge_index, atom_table, rbf_centers,
           W_rbf1, W_rbf2, W_layers, ln_w, ln_b, W_head):
    src = edge_index[0].astype(jnp.int32)
    dst = edge_index[1].astype(jnp.int32)
    px = pos[:, 0]
    py = pos[:, 1]
    pz = pos[:, 2]
    zeros = jnp.zeros((RPT, D), jnp.float32)
    na2 = node_atom.astype(jnp.int32).reshape(N, 1)
    b2 = batch.astype(jnp.int32).reshape(N, 1)

    d2 = _sc_d2(px, py, pz, src, dst)
    ew = _tc_ew(d2.reshape(E, 1), rbf_centers.reshape(1, NB), W_rbf1, W_rbf2)

    x, h = _tc_embed(na2, atom_table, ln_w[0].reshape(1, D), ln_b[0].reshape(1, D))
    for l in range(L):
        aggp = _sc_agg(h, ew, src, dst, zeros)
        nl = (l + 1) % L
        x, h = _tc_update(x, aggp, W_layers[l],
                          ln_w[nl].reshape(1, D), ln_b[nl].reshape(1, D))
    return _tc_head(x, W_head, b2)


# speedup vs baseline: 3.7912x; 1.2886x over previous
import functools
import math

import jax
import jax.numpy as jnp
from jax import lax
from jax.experimental import pallas as pl
from jax.experimental.pallas import tpu as pltpu
import jax.experimental.pallas.tpu_sc as plsc

N = 10000
E = 320000
D = 128
NB = 128
L = 6
NG = 64
NATOM = 64
MAXR = 5.0
AVG_DEG = 32.0

NC = 2
NS = 16
NW = NC * NS
EPT = E // NW
C = 80
K = EPT // C
CA = 40
KA = EPT // CA
NP_ = 10240
RPT = NP_ // NS

_mesh = plsc.VectorSubcoreMesh(core_axis_name="c", subcore_axis_name="s")


@functools.partial(
    pl.kernel,
    out_type=jax.ShapeDtypeStruct((E,), jnp.float32),
    mesh=_mesh,
    scratch_types=[
        pltpu.VMEM((C,), jnp.int32),
        pltpu.VMEM((C,), jnp.int32),
        pltpu.VMEM((6, C), jnp.float32),
        pltpu.VMEM((EPT,), jnp.float32),
        pltpu.SemaphoreType.DMA,
    ],
)
def _sc_d2(px_hbm, py_hbm, pz_hbm, src_hbm, dst_hbm, out_hbm,
           src_v, dst_v, g_v, d2_v, sem):
    c = lax.axis_index("c")
    s = lax.axis_index("s")
    w = c * NS + s
    base = w * EPT

    def chunk(k, _):
        pltpu.sync_copy(src_hbm.at[pl.ds(base + k * C, C)], src_v)
        pltpu.sync_copy(dst_hbm.at[pl.ds(base + k * C, C)], dst_v)
        cps = [
            pltpu.async_copy(px_hbm.at[src_v], g_v.at[0], sem),
            pltpu.async_copy(py_hbm.at[src_v], g_v.at[1], sem),
            pltpu.async_copy(pz_hbm.at[src_v], g_v.at[2], sem),
            pltpu.async_copy(px_hbm.at[dst_v], g_v.at[3], sem),
            pltpu.async_copy(py_hbm.at[dst_v], g_v.at[4], sem),
            pltpu.async_copy(pz_hbm.at[dst_v], g_v.at[5], sem),
        ]
        for cp in cps:
            cp.wait()
        for t in range(C // 16):
            sl = pl.ds(t * 16, 16)
            dx = g_v[0, sl] - g_v[3, sl]
            dy = g_v[1, sl] - g_v[4, sl]
            dz = g_v[2, sl] - g_v[5, sl]
            d2_v[pl.ds(k * C + t * 16, 16)] = dx * dx + dy * dy + dz * dz
        return 0

    lax.fori_loop(0, K, chunk, 0)
    pltpu.sync_copy(d2_v, out_hbm.at[pl.ds(base, EPT)])


@functools.partial(
    pl.kernel,
    out_type=jax.ShapeDtypeStruct((NC, NP_, D), jnp.float32),
    mesh=_mesh,
    scratch_types=[
        pltpu.VMEM_SHARED((NP_, D), jnp.float32),
        pltpu.VMEM((CA,), jnp.int32),
        pltpu.VMEM((CA,), jnp.int32),
        pltpu.VMEM((CA,), jnp.int32),
        pltpu.VMEM((CA,), jnp.int32),
        pltpu.VMEM((CA, D), jnp.float32),
        pltpu.VMEM((CA, D), jnp.float32),
        pltpu.VMEM((CA, D), jnp.float32),
        pltpu.VMEM((CA, D), jnp.float32),
        pltpu.SemaphoreType.DMA,
        pltpu.SemaphoreType.DMA,
        pltpu.SemaphoreType.DMA,
        pltpu.SemaphoreType.DMA,
    ],
)
def _sc_agg(h_hbm, ew_hbm, src_hbm, dst_hbm, zeros_hbm, out_hbm,
            agg_sh, src0, src1, dst0, dst1, msg0, msg1, ew0, ew1,
            sg0, sg1, se0, se1):
    c = lax.axis_index("c")
    s = lax.axis_index("s")
    w = c * NS + s
    base = w * EPT
    row0 = s * RPT
    srcb = (src0, src1)
    dstb = (dst0, dst1)
    msgb = (msg0, msg1)
    ewb = (ew0, ew1)
    sgb = (sg0, sg1)
    seb = (se0, se1)

    pltpu.sync_copy(zeros_hbm, agg_sh.at[pl.ds(row0, RPT)])
    plsc.subcore_barrier()

    def issue(k, b):
        pltpu.sync_copy(src_hbm.at[pl.ds(base + k * CA, CA)], srcb[b])
        pltpu.sync_copy(dst_hbm.at[pl.ds(base + k * CA, CA)], dstb[b])
        pltpu.async_copy(h_hbm.at[srcb[b]], msgb[b], sgb[b])
        pltpu.async_copy(ew_hbm.at[pl.ds(base + k * CA, CA)], ewb[b], seb[b])

    def process(b):
        pltpu.make_async_copy(h_hbm.at[srcb[b]], msgb[b], sgb[b]).wait()
        pltpu.make_async_copy(ew_hbm.at[pl.ds(0, CA)], ewb[b], seb[b]).wait()

        def rowbody(r, _):
            for j in range(D // 16):
                sl = pl.ds(j * 16, 16)
                msgb[b][r, sl] = msgb[b][r, sl] * ewb[b][r, sl]
            return 0

        lax.fori_loop(0, CA, rowbody, 0)
        pltpu.sync_copy(msgb[b], agg_sh.at[dstb[b]], add=True)

    issue(0, 0)
    issue(1, 1)

    def pair(i, _):
        for b in range(2):
            process(b)

            @pl.when(i + 1 < KA // 2)
            def _():
                issue(2 * i + 2 + b, b)
        return 0

    lax.fori_loop(0, KA // 2, pair, 0)
    plsc.subcore_barrier()
    pltpu.sync_copy(agg_sh.at[pl.ds(row0, RPT)],
                    out_hbm.at[c, pl.ds(row0, RPT)])


_RB = 2000
_GN = N // _RB
_EB = 2000
_GE = E // _EB


def _tc_ew_body(d2_ref, cen_ref, w1_ref, w2_ref, out_ref):
    d2 = d2_ref[...]
    dist = jnp.sqrt(d2 + 1e-8)
    cen = cen_ref[...]
    width = MAXR / NB
    rbf = jnp.exp(-((dist - cen) ** 2) * (1.0 / (2.0 * width * width)))
    h1 = jax.nn.silu(jnp.dot(rbf, w1_ref[...], preferred_element_type=jnp.float32))
    out_ref[...] = jnp.dot(h1, w2_ref[...], preferred_element_type=jnp.float32)


def _tc_ew(d2, centers, w1, w2):
    return pl.pallas_call(
        _tc_ew_body,
        grid=(_GE,),
        in_specs=[
            pl.BlockSpec((_EB, 1), lambda i: (i, 0)),
            pl.BlockSpec((1, NB), lambda i: (0, 0)),
            pl.BlockSpec((NB, 64), lambda i: (0, 0)),
            pl.BlockSpec((64, D), lambda i: (0, 0)),
        ],
        out_specs=pl.BlockSpec((_EB, D), lambda i: (i, 0)),
        out_shape=jax.ShapeDtypeStruct((E, D), jnp.float32),
    )(d2, centers, w1, w2)


def _layernorm(x, w, b):
    mu = jnp.mean(x, axis=-1, keepdims=True)
    var = jnp.var(x, axis=-1, keepdims=True)
    return (x - mu) / jnp.sqrt(var + 1e-5) * w + b


def _tc_embed_body(na_ref, tab_ref, lnw_ref, lnb_ref, x_ref, h_ref):
    na = na_ref[...]
    ids = lax.broadcasted_iota(jnp.int32, (_RB, NATOM), 1)
    onehot = (na == ids).astype(jnp.float32)
    x = jnp.dot(onehot, tab_ref[...], preferred_element_type=jnp.float32)
    x_ref[...] = x
    h_ref[...] = _layernorm(x, lnw_ref[...], lnb_ref[...])


def _tc_embed(na, tab, lnw0, lnb0):
    return pl.pallas_call(
        _tc_embed_body,
        grid=(_GN,),
        in_specs=[
            pl.BlockSpec((_RB, 1), lambda i: (i, 0)),
            pl.BlockSpec((NATOM, D), lambda i: (0, 0)),
            pl.BlockSpec((1, D), lambda i: (0, 0)),
            pl.BlockSpec((1, D), lambda i: (0, 0)),
        ],
        out_specs=[
            pl.BlockSpec((_RB, D), lambda i: (i, 0)),
            pl.BlockSpec((_RB, D), lambda i: (i, 0)),
        ],
        out_shape=[
            jax.ShapeDtypeStruct((N, D), jnp.float32),
            jax.ShapeDtypeStruct((N, D), jnp.float32),
        ],
    )(na, tab, lnw0, lnb0)


def _tc_update_body(x_ref, agg_ref, w_ref, lnw_ref, lnb_ref, xn_ref, hn_ref):
    agg = (agg_ref[0] + agg_ref[1]) * (1.0 / math.sqrt(AVG_DEG))
    up = jnp.dot(agg, w_ref[...], preferred_element_type=jnp.float32)
    xn = x_ref[...] + jax.nn.silu(up)
    xn_ref[...] = xn
    hn_ref[...] = _layernorm(xn, lnw_ref[...], lnb_ref[...])


def _tc_update(x, aggp, w, lnw, lnb):
    return pl.pallas_call(
        _tc_update_body,
        grid=(_GN,),
        in_specs=[
            pl.BlockSpec((_RB, D), lambda i: (i, 0)),
            pl.BlockSpec((NC, _RB, D), lambda i: (0, i, 0)),
            pl.BlockSpec((D, D), lambda i: (0, 0)),
            pl.BlockSpec((1, D), lambda i: (0, 0)),
            pl.BlockSpec((1, D), lambda i: (0, 0)),
        ],
        out_specs=[
            pl.BlockSpec((_RB, D), lambda i: (i, 0)),
            pl.BlockSpec((_RB, D), lambda i: (i, 0)),
        ],
        out_shape=[
            jax.ShapeDtypeStruct((N, D), jnp.float32),
            jax.ShapeDtypeStruct((N, D), jnp.float32),
        ],
    )(x, aggp, w, lnw, lnb)


def _tc_head_body(x_ref, wh_ref, b_ref, out_ref):
    i = pl.program_id(0)

    @pl.when(i == 0)
    def _():
        out_ref[...] = jnp.zeros_like(out_ref)

    e_node = jnp.dot(x_ref[...], wh_ref[...], preferred_element_type=jnp.float32)
    ids = lax.broadcasted_iota(jnp.int32, (_RB, NG), 1)
    onehot = (b_ref[...] == ids).astype(jnp.float32)
    out_ref[...] += lax.dot_general(
        onehot, e_node, (((0,), (0,)), ((), ())),
        preferred_element_type=jnp.float32)


def _tc_head(x, wh, batch2):
    return pl.pallas_call(
        _tc_head_body,
        grid=(_GN,),
        in_specs=[
            pl.BlockSpec((_RB, D), lambda i: (i, 0)),
            pl.BlockSpec((D, 1), lambda i: (0, 0)),
            pl.BlockSpec((_RB, 1), lambda i: (i, 0)),
        ],
        out_specs=pl.BlockSpec((NG, 1), lambda i: (0, 0)),
        out_shape=jax.ShapeDtypeStruct((NG, 1), jnp.float32),
    )(x, wh, batch2)


def kernel(pos, node_atom, batch, edge_index, atom_table, rbf_centers,
           W_rbf1, W_rbf2, W_layers, ln_w, ln_b, W_head):
    src = edge_index[0].astype(jnp.int32)
    dst = edge_index[1].astype(jnp.int32)
    px = pos[:, 0]
    py = pos[:, 1]
    pz = pos[:, 2]
    zeros = jnp.zeros((RPT, D), jnp.float32)
    na2 = node_atom.astype(jnp.int32).reshape(N, 1)
    b2 = batch.astype(jnp.int32).reshape(N, 1)

    d2 = _sc_d2(px, py, pz, src, dst)
    ew = _tc_ew(d2.reshape(E, 1), rbf_centers.reshape(1, NB), W_rbf1, W_rbf2)

    x, h = _tc_embed(na2, atom_table, ln_w[0].reshape(1, D), ln_b[0].reshape(1, D))
    for l in range(L):
        aggp = _sc_agg(h, ew, src, dst, zeros)
        nl = (l + 1) % L
        x, h = _tc_update(x, aggp, W_layers[l],
                          ln_w[nl].reshape(1, D), ln_b[nl].reshape(1, D))
    return _tc_head(x, W_head, b2)


# speedup vs baseline: 5.5855x; 1.4733x over previous
import functools
import math

import jax
import jax.numpy as jnp
from jax import lax
from jax.experimental import pallas as pl
from jax.experimental.pallas import tpu as pltpu
import jax.experimental.pallas.tpu_sc as plsc

N = 10000
E = 320000
D = 128
NB = 128
L = 6
NG = 64
NATOM = 64
MAXR = 5.0
AVG_DEG = 32.0

NC = 2
NS = 16
NW = NC * NS
EPT = E // NW
C = 80
K = EPT // C
CA = 40
KA = EPT // CA
NP_ = 10240
RPT = NP_ // NS

_mesh = plsc.VectorSubcoreMesh(core_axis_name="c", subcore_axis_name="s")


@functools.partial(
    pl.kernel,
    out_type=jax.ShapeDtypeStruct((E,), jnp.float32),
    mesh=_mesh,
    scratch_types=[
        pltpu.VMEM((C,), jnp.int32),
        pltpu.VMEM((C,), jnp.int32),
        pltpu.VMEM((6, C), jnp.float32),
        pltpu.VMEM((EPT,), jnp.float32),
        pltpu.SemaphoreType.DMA,
    ],
)
def _sc_d2(px_hbm, py_hbm, pz_hbm, src_hbm, dst_hbm, out_hbm,
           src_v, dst_v, g_v, d2_v, sem):
    c = lax.axis_index("c")
    s = lax.axis_index("s")
    w = c * NS + s
    base = w * EPT

    def chunk(k, _):
        pltpu.sync_copy(src_hbm.at[pl.ds(base + k * C, C)], src_v)
        pltpu.sync_copy(dst_hbm.at[pl.ds(base + k * C, C)], dst_v)
        cps = [
            pltpu.async_copy(px_hbm.at[src_v], g_v.at[0], sem),
            pltpu.async_copy(py_hbm.at[src_v], g_v.at[1], sem),
            pltpu.async_copy(pz_hbm.at[src_v], g_v.at[2], sem),
            pltpu.async_copy(px_hbm.at[dst_v], g_v.at[3], sem),
            pltpu.async_copy(py_hbm.at[dst_v], g_v.at[4], sem),
            pltpu.async_copy(pz_hbm.at[dst_v], g_v.at[5], sem),
        ]
        for cp in cps:
            cp.wait()
        for t in range(C // 16):
            sl = pl.ds(t * 16, 16)
            dx = g_v[0, sl] - g_v[3, sl]
            dy = g_v[1, sl] - g_v[4, sl]
            dz = g_v[2, sl] - g_v[5, sl]
            d2_v[pl.ds(k * C + t * 16, 16)] = dx * dx + dy * dy + dz * dz
        return 0

    lax.fori_loop(0, K, chunk, 0)
    pltpu.sync_copy(d2_v, out_hbm.at[pl.ds(base, EPT)])


@functools.partial(
    pl.kernel,
    out_type=jax.ShapeDtypeStruct((NC, NP_, D), jnp.float32),
    mesh=_mesh,
    scratch_types=[
        pltpu.VMEM_SHARED((NP_, D), jnp.float32),
        pltpu.VMEM((EPT,), jnp.int32),
        pltpu.VMEM((EPT,), jnp.int32),
        pltpu.VMEM((CA, D), jnp.float32),
        pltpu.VMEM((CA, D), jnp.float32),
        pltpu.VMEM((CA, D), jnp.float32),
        pltpu.VMEM((CA, D), jnp.float32),
        pltpu.SemaphoreType.DMA,
        pltpu.SemaphoreType.DMA,
        pltpu.SemaphoreType.DMA,
        pltpu.SemaphoreType.DMA,
    ],
)
def _sc_agg(h_hbm, ew_hbm, src_hbm, dst_hbm, zeros_hbm, out_hbm,
            agg_sh, srcall, dstall, msg0, msg1, ew0, ew1,
            sg0, sg1, se0, se1):
    c = lax.axis_index("c")
    s = lax.axis_index("s")
    w = c * NS + s
    base = w * EPT
    row0 = s * RPT
    msgb = (msg0, msg1)
    ewb = (ew0, ew1)
    sgb = (sg0, sg1)
    seb = (se0, se1)

    pltpu.sync_copy(zeros_hbm, agg_sh.at[pl.ds(row0, RPT)])
    pltpu.sync_copy(src_hbm.at[pl.ds(base, EPT)], srcall)
    pltpu.sync_copy(dst_hbm.at[pl.ds(base, EPT)], dstall)
    plsc.subcore_barrier()

    def issue(k, b):
        pltpu.async_copy(h_hbm.at[srcall.at[pl.ds(k * CA, CA)]], msgb[b], sgb[b])
        pltpu.async_copy(ew_hbm.at[pl.ds(base + k * CA, CA)], ewb[b], seb[b])

    def process(k, b):
        pltpu.make_async_copy(h_hbm.at[srcall.at[pl.ds(0, CA)]], msgb[b], sgb[b]).wait()
        pltpu.make_async_copy(ew_hbm.at[pl.ds(0, CA)], ewb[b], seb[b]).wait()

        def rowbody(r, _):
            for j in range(D // 16):
                sl = pl.ds(j * 16, 16)
                msgb[b][r, sl] = msgb[b][r, sl] * ewb[b][r, sl]
            return 0

        lax.fori_loop(0, CA, rowbody, 0)
        pltpu.sync_copy(msgb[b], agg_sh.at[dstall.at[pl.ds(k * CA, CA)]], add=True)

    issue(0, 0)
    issue(1, 1)

    def pair(i, _):
        for b in range(2):
            process(2 * i + b, b)

            @pl.when(i + 1 < KA // 2)
            def _():
                issue(2 * i + 2 + b, b)
        return 0

    lax.fori_loop(0, KA // 2, pair, 0)
    plsc.subcore_barrier()
    pltpu.sync_copy(agg_sh.at[pl.ds(row0, RPT)],
                    out_hbm.at[c, pl.ds(row0, RPT)])


_RB = 2000
_GN = N // _RB
_EB = 2000
_GE = E // _EB


def _tc_ew_body(d2_ref, cen_ref, w1_ref, w2_ref, out_ref):
    d2 = d2_ref[...]
    dist = jnp.sqrt(d2 + 1e-8)
    cen = cen_ref[...]
    width = MAXR / NB
    rbf = jnp.exp(-((dist - cen) ** 2) * (1.0 / (2.0 * width * width)))
    h1 = jax.nn.silu(jnp.dot(rbf, w1_ref[...], preferred_element_type=jnp.float32))
    out_ref[...] = jnp.dot(h1, w2_ref[...], preferred_element_type=jnp.float32)


def _tc_ew(d2, centers, w1, w2):
    return pl.pallas_call(
        _tc_ew_body,
        grid=(_GE,),
        in_specs=[
            pl.BlockSpec((_EB, 1), lambda i: (i, 0)),
            pl.BlockSpec((1, NB), lambda i: (0, 0)),
            pl.BlockSpec((NB, 64), lambda i: (0, 0)),
            pl.BlockSpec((64, D), lambda i: (0, 0)),
        ],
        out_specs=pl.BlockSpec((_EB, D), lambda i: (i, 0)),
        out_shape=jax.ShapeDtypeStruct((E, D), jnp.float32),
    )(d2, centers, w1, w2)


def _layernorm(x, w, b):
    mu = jnp.mean(x, axis=-1, keepdims=True)
    var = jnp.var(x, axis=-1, keepdims=True)
    return (x - mu) / jnp.sqrt(var + 1e-5) * w + b


def _tc_embed_body(na_ref, tab_ref, lnw_ref, lnb_ref, x_ref, h_ref):
    na = na_ref[...]
    ids = lax.broadcasted_iota(jnp.int32, (_RB, NATOM), 1)
    onehot = (na == ids).astype(jnp.float32)
    x = jnp.dot(onehot, tab_ref[...], preferred_element_type=jnp.float32)
    x_ref[...] = x
    h_ref[...] = _layernorm(x, lnw_ref[...], lnb_ref[...])


def _tc_embed(na, tab, lnw0, lnb0):
    return pl.pallas_call(
        _tc_embed_body,
        grid=(_GN,),
        in_specs=[
            pl.BlockSpec((_RB, 1), lambda i: (i, 0)),
            pl.BlockSpec((NATOM, D), lambda i: (0, 0)),
            pl.BlockSpec((1, D), lambda i: (0, 0)),
            pl.BlockSpec((1, D), lambda i: (0, 0)),
        ],
        out_specs=[
            pl.BlockSpec((_RB, D), lambda i: (i, 0)),
            pl.BlockSpec((_RB, D), lambda i: (i, 0)),
        ],
        out_shape=[
            jax.ShapeDtypeStruct((N, D), jnp.float32),
            jax.ShapeDtypeStruct((N, D), jnp.float32),
        ],
    )(na, tab, lnw0, lnb0)


def _tc_update_body(x_ref, agg_ref, w_ref, lnw_ref, lnb_ref, xn_ref, hn_ref):
    agg = (agg_ref[0] + agg_ref[1]) * (1.0 / math.sqrt(AVG_DEG))
    up = jnp.dot(agg, w_ref[...], preferred_element_type=jnp.float32)
    xn = x_ref[...] + jax.nn.silu(up)
    xn_ref[...] = xn
    hn_ref[...] = _layernorm(xn, lnw_ref[...], lnb_ref[...])


def _tc_update(x, aggp, w, lnw, lnb):
    return pl.pallas_call(
        _tc_update_body,
        grid=(_GN,),
        in_specs=[
            pl.BlockSpec((_RB, D), lambda i: (i, 0)),
            pl.BlockSpec((NC, _RB, D), lambda i: (0, i, 0)),
            pl.BlockSpec((D, D), lambda i: (0, 0)),
            pl.BlockSpec((1, D), lambda i: (0, 0)),
            pl.BlockSpec((1, D), lambda i: (0, 0)),
        ],
        out_specs=[
            pl.BlockSpec((_RB, D), lambda i: (i, 0)),
            pl.BlockSpec((_RB, D), lambda i: (i, 0)),
        ],
        out_shape=[
            jax.ShapeDtypeStruct((N, D), jnp.float32),
            jax.ShapeDtypeStruct((N, D), jnp.float32),
        ],
    )(x, aggp, w, lnw, lnb)


def _tc_head_body(x_ref, wh_ref, b_ref, out_ref):
    i = pl.program_id(0)

    @pl.when(i == 0)
    def _():
        out_ref[...] = jnp.zeros_like(out_ref)

    e_node = jnp.dot(x_ref[...], wh_ref[...], preferred_element_type=jnp.float32)
    ids = lax.broadcasted_iota(jnp.int32, (_RB, NG), 1)
    onehot = (b_ref[...] == ids).astype(jnp.float32)
    out_ref[...] += lax.dot_general(
        onehot, e_node, (((0,), (0,)), ((), ())),
        preferred_element_type=jnp.float32)


def _tc_head(x, wh, batch2):
    return pl.pallas_call(
        _tc_head_body,
        grid=(_GN,),
        in_specs=[
            pl.BlockSpec((_RB, D), lambda i: (i, 0)),
            pl.BlockSpec((D, 1), lambda i: (0, 0)),
            pl.BlockSpec((_RB, 1), lambda i: (i, 0)),
        ],
        out_specs=pl.BlockSpec((NG, 1), lambda i: (0, 0)),
        out_shape=jax.ShapeDtypeStruct((NG, 1), jnp.float32),
    )(x, wh, batch2)


def kernel(pos, node_atom, batch, edge_index, atom_table, rbf_centers,
           W_rbf1, W_rbf2, W_layers, ln_w, ln_b, W_head):
    src = edge_index[0].astype(jnp.int32)
    dst = edge_index[1].astype(jnp.int32)
    px = pos[:, 0]
    py = pos[:, 1]
    pz = pos[:, 2]
    zeros = jnp.zeros((RPT, D), jnp.float32)
    na2 = node_atom.astype(jnp.int32).reshape(N, 1)
    b2 = batch.astype(jnp.int32).reshape(N, 1)

    d2 = _sc_d2(px, py, pz, src, dst)
    ew = _tc_ew(d2.reshape(E, 1), rbf_centers.reshape(1, NB), W_rbf1, W_rbf2)

    x, h = _tc_embed(na2, atom_table, ln_w[0].reshape(1, D), ln_b[0].reshape(1, D))
    for l in range(L):
        aggp = _sc_agg(h, ew, src, dst, zeros)
        nl = (l + 1) % L
        x, h = _tc_update(x, aggp, W_layers[l],
                          ln_w[nl].reshape(1, D), ln_b[nl].reshape(1, D))
    return _tc_head(x, W_head, b2)
